# Initial kernel scaffold; baseline (speedup 1.0000x reference)
import jax
import jax.numpy as jnp
from jax.experimental import pallas as pl


def kernel(x, copy_tensor):
    raise NotImplementedError("write your pallas kernel here")


# speedup vs baseline: 463.4891x; 463.4891x over previous
import functools

import jax
import jax.numpy as jnp
from jax import lax
from jax.experimental import pallas as pl
from jax.experimental.pallas import tpu as pltpu
from jax.experimental.pallas import tpu_sc as plsc

B = 8192
F = 2048
NE = 21
L = 16

_info = plsc.get_sparse_core_info()
NC, NS = _info.num_cores, _info.num_subcores
NW = NC * NS
ROWS_W = B // NW
CH = 8
NCHUNK = ROWS_W // CH
CW = CH * F
VPC = CW // L
VPR = F // L

_mesh = plsc.VectorSubcoreMesh(core_axis_name="c", subcore_axis_name="s")


@functools.partial(
    pl.kernel,
    mesh=_mesh,
    out_type=jax.ShapeDtypeStruct((B * F,), jnp.float32),
    scratch_types=[
        pltpu.VMEM((F * NE,), jnp.float32),
        pltpu.VMEM((CW,), jnp.float32),
    ],
    compiler_params=pltpu.CompilerParams(needs_layout_passes=False),
)
def _sc_lerp(x_hbm, ct_hbm, out_hbm, t_v, xb):
    wid = lax.axis_index("s") * NC + lax.axis_index("c")
    base = wid * (ROWS_W * F)

    pltpu.sync_copy(ct_hbm, t_v)

    lane = jnp.arange(L, dtype=jnp.int32)
    lane21 = lane * NE

    def body(j, _):
        off = j * L
        fb21 = lax.rem(j, VPR) * (L * NE)
        xv = xb[pl.ds(off, L)]
        s = xv + 10.0
        li = jnp.minimum(jnp.maximum(s.astype(jnp.int32), 0), 19)
        gidx = lane21 + fb21 + li
        lo = plsc.load_gather(t_v, [gidx])
        hi = plsc.load_gather(t_v, [gidx + 1])
        frac = s - li.astype(jnp.float32)
        xb[pl.ds(off, L)] = lo + frac * (hi - lo)
        return _

    for c in range(NCHUNK):
        off = base + c * CW
        pltpu.sync_copy(x_hbm.at[pl.ds(off, CW)], xb)
        lax.fori_loop(0, VPC, body, None)
        pltpu.sync_copy(xb, out_hbm.at[pl.ds(off, CW)])


def kernel(x, copy_tensor):
    out = _sc_lerp(x.reshape(-1), copy_tensor.reshape(-1))
    return out.reshape(x.shape)


# speedup vs baseline: 1565.8053x; 3.3783x over previous
import functools

import jax
import jax.numpy as jnp
from jax import lax
from jax.experimental import pallas as pl
from jax.experimental.pallas import tpu as pltpu
from jax.experimental.pallas import tpu_sc as plsc

B = 8192
F = 2048
NE = 21
L = 16

_info = plsc.get_sparse_core_info()
NC, NS = _info.num_cores, _info.num_subcores
NW = NC * NS
ROWS_W = B // NW
CH = 8
NCHUNK = ROWS_W // CH
CW = CH * F
VPC = CW // L
VPR = F // L
UNROLL = 4

_mesh = plsc.VectorSubcoreMesh(core_axis_name="c", subcore_axis_name="s")


@functools.partial(
    pl.kernel,
    mesh=_mesh,
    out_type=jax.ShapeDtypeStruct((B * F,), jnp.float32),
    scratch_types=[
        pltpu.VMEM((F * NE,), jnp.float32),
        pltpu.VMEM((CW,), jnp.float32),
        pltpu.VMEM((CW,), jnp.float32),
        pltpu.VMEM((CW,), jnp.float32),
        pltpu.VMEM((CW,), jnp.float32),
        pltpu.SemaphoreType.DMA,
        pltpu.SemaphoreType.DMA,
        pltpu.SemaphoreType.DMA,
        pltpu.SemaphoreType.DMA,
    ],
    compiler_params=pltpu.CompilerParams(needs_layout_passes=False),
)
def _sc_lerp(x_hbm, ct_hbm, out_hbm, t_v, in0, in1, ob0, ob1,
             sf0, sf1, sd0, sd1):
    wid = lax.axis_index("s") * NC + lax.axis_index("c")
    base = wid * (ROWS_W * F)

    pltpu.sync_copy(ct_hbm, t_v)

    ins, obs = [in0, in1], [ob0, ob1]
    sfs, sds = [sf0, sf1], [sd0, sd1]
    lane21 = jnp.arange(L, dtype=jnp.int32) * NE

    def compute(xb, ob):
        @plsc.parallel_loop(0, VPC, unroll=UNROLL)
        def body(j):
            off = j * L
            fb21 = lax.rem(j, VPR) * (L * NE)
            xv = xb[pl.ds(off, L)]
            s = xv + 10.0
            li = jnp.minimum(jnp.maximum(s.astype(jnp.int32), 0), 19)
            gidx = lane21 + fb21 + li
            lo = plsc.load_gather(t_v, [gidx])
            hi = plsc.load_gather(t_v, [gidx + 1])
            frac = s - li.astype(jnp.float32)
            ob[pl.ds(off, L)] = lo + frac * (hi - lo)

    fills = [None] * NCHUNK
    drains = [None] * NCHUNK
    fills[0] = pltpu.async_copy(x_hbm.at[pl.ds(base, CW)], in0, sf0)
    for c in range(NCHUNK):
        b = c & 1
        if c + 1 < NCHUNK:
            fills[c + 1] = pltpu.async_copy(
                x_hbm.at[pl.ds(base + (c + 1) * CW, CW)], ins[b ^ 1],
                sfs[b ^ 1])
        fills[c].wait()
        if c >= 2:
            drains[c - 2].wait()
        compute(ins[b], obs[b])
        drains[c] = pltpu.async_copy(
            obs[b], out_hbm.at[pl.ds(base + c * CW, CW)], sds[b])
    drains[NCHUNK - 2].wait()
    drains[NCHUNK - 1].wait()


def kernel(x, copy_tensor):
    out = _sc_lerp(x.reshape(-1), copy_tensor.reshape(-1))
    return out.reshape(x.shape)


# speedup vs baseline: 1597.5162x; 1.0203x over previous
import functools

import jax
import jax.numpy as jnp
from jax import lax
from jax.experimental import pallas as pl
from jax.experimental.pallas import tpu as pltpu
from jax.experimental.pallas import tpu_sc as plsc

B = 8192
F = 2048
NE = 21
L = 16

_info = plsc.get_sparse_core_info()
NC, NS = _info.num_cores, _info.num_subcores
NW = NC * NS
ROWS_W = B // NW
CH = 8
NCHUNK = ROWS_W // CH
CW = CH * F
VPC = CW // L
VPR = F // L
UNROLL = 8

_mesh = plsc.VectorSubcoreMesh(core_axis_name="c", subcore_axis_name="s")


@functools.partial(
    pl.kernel,
    mesh=_mesh,
    out_type=jax.ShapeDtypeStruct((B * F,), jnp.float32),
    scratch_types=[
        pltpu.VMEM((F * NE,), jnp.float32),
        pltpu.VMEM((CW,), jnp.float32),
        pltpu.VMEM((CW,), jnp.float32),
        pltpu.VMEM((CW,), jnp.float32),
        pltpu.VMEM((CW,), jnp.float32),
        pltpu.SemaphoreType.DMA,
        pltpu.SemaphoreType.DMA,
        pltpu.SemaphoreType.DMA,
        pltpu.SemaphoreType.DMA,
    ],
    compiler_params=pltpu.CompilerParams(needs_layout_passes=False),
)
def _sc_lerp(x_hbm, ct_hbm, out_hbm, t_v, in0, in1, ob0, ob1,
             sf0, sf1, sd0, sd1):
    wid = lax.axis_index("s") * NC + lax.axis_index("c")
    base = wid * (ROWS_W * F)

    pltpu.sync_copy(ct_hbm, t_v)

    ins, obs = [in0, in1], [ob0, ob1]
    sfs, sds = [sf0, sf1], [sd0, sd1]
    lane21 = jnp.arange(L, dtype=jnp.int32) * NE

    def compute(xb, ob):
        @plsc.parallel_loop(0, VPC, unroll=UNROLL)
        def body(j):
            off = j * L
            fb21 = lax.rem(j, VPR) * (L * NE)
            xv = xb[pl.ds(off, L)]
            s = xv + 10.0
            li = jnp.minimum(jnp.maximum(s.astype(jnp.int32), 0), 19)
            gidx = lane21 + fb21 + li
            lo = plsc.load_gather(t_v, [gidx])
            hi = plsc.load_gather(t_v, [gidx + 1])
            frac = s - li.astype(jnp.float32)
            ob[pl.ds(off, L)] = lo + frac * (hi - lo)

    fills = [None] * NCHUNK
    drains = [None] * NCHUNK
    fills[0] = pltpu.async_copy(x_hbm.at[pl.ds(base, CW)], in0, sf0)
    for c in range(NCHUNK):
        b = c & 1
        if c + 1 < NCHUNK:
            fills[c + 1] = pltpu.async_copy(
                x_hbm.at[pl.ds(base + (c + 1) * CW, CW)], ins[b ^ 1],
                sfs[b ^ 1])
        fills[c].wait()
        if c >= 2:
            drains[c - 2].wait()
        compute(ins[b], obs[b])
        drains[c] = pltpu.async_copy(
            obs[b], out_hbm.at[pl.ds(base + c * CW, CW)], sds[b])
    drains[NCHUNK - 2].wait()
    drains[NCHUNK - 1].wait()


def kernel(x, copy_tensor):
    out = _sc_lerp(x.reshape(-1), copy_tensor.reshape(-1))
    return out.reshape(x.shape)
